# baseline (speedup 1.0000x reference)
import jax
import jax.numpy as jnp
from jax.experimental import pallas as pl

N_ROWS = 1_600_000
BLOCK = 16_000


def _net_block(z_ref, a_ref, c_ref, o_ref):
    zb = z_ref[...]
    d = jnp.dot(zb, a_ref[...], preferred_element_type=jnp.float32)
    o_ref[...] = jax.nn.sigmoid(d + c_ref[...])


def kernel(z, W, b):
    wd = W[0] - W[1]
    bd = b[0] - b[1]
    a = jnp.stack([wd, -wd], axis=1)
    c = jnp.stack([bd, -bd]).reshape(1, 2)
    n = z.shape[0]
    grid = (n // BLOCK,)
    return pl.pallas_call(
        _net_block,
        grid=grid,
        in_specs=[
            pl.BlockSpec((BLOCK, 32), lambda i: (i, 0)),
            pl.BlockSpec((32, 2), lambda i: (0, 0)),
            pl.BlockSpec((1, 2), lambda i: (0, 0)),
        ],
        out_specs=pl.BlockSpec((BLOCK, 2), lambda i: (i, 0)),
        out_shape=jax.ShapeDtypeStruct((n, 2), jnp.float32),
    )(z, a, c)
